# baseline (speedup 1.0000x reference)
import functools

import jax
import jax.numpy as jnp
import numpy as np
from jax import lax
from jax.experimental import pallas as pl
from jax.experimental.pallas import tpu as pltpu

_K = 3
_TINY = np.float32(np.finfo(np.float32).tiny)
_ROT_A = (13, 15, 26, 6)
_ROT_B = (17, 29, 16, 24)


def _threefry_rounds(x0, x1, rots):
    for r in rots:
        x0 = x0 + x1
        x1 = lax.shift_left(x1, np.int32(r)) | lax.shift_right_logical(
            x1, np.int32(32 - r)
        )
        x1 = x1 ^ x0
    return x0, x1


def _uniform_from_index(idx):
    ks1 = np.int32(42)
    ks2 = np.int32(0x1BD11BDA ^ 42)
    x1 = idx
    x0 = x1
    x1 = (
        lax.shift_left(x1, np.int32(13))
        | lax.shift_right_logical(x1, np.int32(19))
    ) ^ x0
    x0, x1 = _threefry_rounds(x0, x1, _ROT_A[1:])
    x0, x1 = x0 + ks1, x1 + np.int32(ks2 + 1)
    x0, x1 = _threefry_rounds(x0, x1, _ROT_B)
    x0, x1 = x0 + ks2, x1 + np.int32(2)
    x0, x1 = _threefry_rounds(x0, x1, _ROT_A)
    x0, x1 = x0, x1 + np.int32(ks1 + 3)
    x0, x1 = _threefry_rounds(x0, x1, _ROT_B)
    x0, x1 = x0 + ks1, x1 + np.int32(ks2 + 4)
    x0, x1 = _threefry_rounds(x0, x1, _ROT_A)
    x0, x1 = x0 + ks2, x1 + np.int32(5)
    bits = x0 ^ x1
    float_bits = lax.shift_right_logical(bits, np.int32(9)) | np.int32(0x3F800000)
    f = lax.bitcast_convert_type(float_bits, jnp.float32) - np.float32(1.0)
    return jnp.maximum(f, _TINY)


def _pool_kernel(x_ref, pat_ref, o_ref, *, oh, ow, rh):
    c = pl.program_id(0)
    rs = pl.program_id(1)
    row0 = rs * np.int32(rh)
    pattern = pat_ref[...]
    slab = x_ref[0, pl.ds(row0, rh + 8), :]

    def win(dy, dx):
        return slab[dy : dy + rh, dx : dx + ow]

    colmax = None
    for dx in range(_K):
        v = slab[:, dx : dx + ow]
        colmax = v if colmax is None else jnp.maximum(colmax, v)
    maxp = None
    for dy in range(_K):
        v = colmax[dy : dy + rh]
        maxp = v if maxp is None else jnp.maximum(maxp, v)
    zero_den = maxp <= np.float32(0.0)

    base = pattern + c * np.int32(oh * ow * 9)

    best_score = jnp.full((rh, ow), -jnp.inf, jnp.float32)
    best_val = jnp.zeros((rh, ow), jnp.float32)
    for j in range(9):
        dy, dx = divmod(j, _K)
        p = win(dy, dx)
        u = _uniform_from_index(base + np.int32(j))
        score = jnp.where(zero_den, u, jnp.maximum(p, 0.0) / -jnp.log(u))
        take = score > best_score
        best_score = jnp.where(take, score, best_score)
        best_val = jnp.where(take, p, best_val)
    o_ref[0] = best_val


@jax.jit
def kernel(x):
    B, C, H, W = x.shape
    oh = H - _K + 1
    ow = W - _K + 1
    N = B * C
    S = 4 if oh >= 128 else 1
    rh = -(-(-(-oh // S)) // 8) * 8
    hpad = -(-(S * rh + _K - 1) // 8) * 8
    yy = np.arange(oh, dtype=np.int64)[:, None]
    xxx = np.arange(ow, dtype=np.int64)[None, :]
    pattern = jnp.asarray(((yy * ow + xxx) * 9 + 42).astype(np.int32))
    out = pl.pallas_call(
        functools.partial(_pool_kernel, oh=oh, ow=ow, rh=rh),
        grid=(N, S),
        in_specs=[
            pl.BlockSpec((1, hpad, W), lambda c, rs: (c, 0, 0)),
            pl.BlockSpec((rh, ow), lambda c, rs: (rs, 0)),
        ],
        out_specs=pl.BlockSpec((1, rh, ow), lambda c, rs: (c, rs, 0)),
        out_shape=jax.ShapeDtypeStruct((N, oh, ow), jnp.float32),
        compiler_params=pltpu.CompilerParams(
            dimension_semantics=("parallel", "arbitrary")
        ),
    )(x.reshape(N, H, W), pattern)
    return out.reshape(B, C, oh, ow)


# speedup vs baseline: 1.0075x; 1.0075x over previous
import functools

import jax
import jax.numpy as jnp
import numpy as np
from jax import lax
from jax.experimental import pallas as pl
from jax.experimental.pallas import tpu as pltpu

_K = 3
_TINY = np.float32(np.finfo(np.float32).tiny)
_ROT_A = (13, 15, 26, 6)
_ROT_B = (17, 29, 16, 24)


def _threefry_rounds(x0, x1, rots):
    for r in rots:
        x0 = x0 + x1
        x1 = lax.shift_left(x1, np.int32(r)) | lax.shift_right_logical(
            x1, np.int32(32 - r)
        )
        x1 = x1 ^ x0
    return x0, x1


def _uniform_from_index(idx):
    ks1 = np.int32(42)
    ks2 = np.int32(0x1BD11BDA ^ 42)
    x1 = idx
    x0 = x1
    x1 = (
        lax.shift_left(x1, np.int32(13))
        | lax.shift_right_logical(x1, np.int32(19))
    ) ^ x0
    x0, x1 = _threefry_rounds(x0, x1, _ROT_A[1:])
    x0, x1 = x0 + ks1, x1 + np.int32(ks2 + 1)
    x0, x1 = _threefry_rounds(x0, x1, _ROT_B)
    x0, x1 = x0 + ks2, x1 + np.int32(2)
    x0, x1 = _threefry_rounds(x0, x1, _ROT_A)
    x0, x1 = x0, x1 + np.int32(ks1 + 3)
    x0, x1 = _threefry_rounds(x0, x1, _ROT_B)
    x0, x1 = x0 + ks1, x1 + np.int32(ks2 + 4)
    x0, x1 = _threefry_rounds(x0, x1, _ROT_A)
    x0, x1 = x0 + ks2, x1 + np.int32(5)
    bits = x0 ^ x1
    float_bits = lax.shift_right_logical(bits, np.int32(9)) | np.int32(0x3F800000)
    f = lax.bitcast_convert_type(float_bits, jnp.float32) - np.float32(1.0)
    return jnp.maximum(f, _TINY)


def _pool_kernel(x_ref, pat_ref, o_ref, *, oh, ow, rh):
    c = pl.program_id(0)
    rs = pl.program_id(1)
    row0 = rs * np.int32(rh)
    pattern = pat_ref[...]
    slab = x_ref[0, pl.ds(row0, rh + 8), :]

    def win(dy, dx):
        return slab[dy : dy + rh, dx : dx + ow]

    maxp = None
    for dy in range(_K):
        for dx in range(_K):
            v = win(dy, dx)
            maxp = v if maxp is None else jnp.maximum(maxp, v)
    zero_den = maxp <= np.float32(0.0)

    base = pattern + c * np.int32(oh * ow * 9)

    best_score = jnp.full((rh, ow), -jnp.inf, jnp.float32)
    best_val = jnp.zeros((rh, ow), jnp.float32)
    for j in range(9):
        dy, dx = divmod(j, _K)
        p = win(dy, dx)
        u = _uniform_from_index(base + np.int32(j))
        score = jnp.where(zero_den, u, jnp.maximum(p, 0.0) / -jnp.log(u))
        take = score > best_score
        best_score = jnp.where(take, score, best_score)
        best_val = jnp.where(take, p, best_val)
    o_ref[0] = best_val


@jax.jit
def kernel(x):
    B, C, H, W = x.shape
    oh = H - _K + 1
    ow = W - _K + 1
    N = B * C
    S = 4 if oh >= 128 else 1
    rh = -(-(-(-oh // S)) // 8) * 8
    hpad = -(-(S * rh + _K - 1) // 8) * 8
    yy = np.arange(oh, dtype=np.int64)[:, None]
    xxx = np.arange(ow, dtype=np.int64)[None, :]
    pattern = jnp.asarray(((yy * ow + xxx) * 9 + 42).astype(np.int32))
    out = pl.pallas_call(
        functools.partial(_pool_kernel, oh=oh, ow=ow, rh=rh),
        grid=(N, S),
        in_specs=[
            pl.BlockSpec((1, hpad, W), lambda c, rs: (c, 0, 0)),
            pl.BlockSpec((rh, ow), lambda c, rs: (rs, 0)),
        ],
        out_specs=pl.BlockSpec((1, rh, ow), lambda c, rs: (c, rs, 0)),
        out_shape=jax.ShapeDtypeStruct((N, oh, ow), jnp.float32),
        compiler_params=pltpu.CompilerParams(
            dimension_semantics=("parallel", "arbitrary")
        ),
    )(x.reshape(N, H, W), pattern)
    return out.reshape(B, C, oh, ow)
